# Initial kernel scaffold; baseline (speedup 1.0000x reference)
import jax
import jax.numpy as jnp
from jax.experimental import pallas as pl


def kernel(inputs, embedding_weight):
    raise NotImplementedError("write your pallas kernel here")


# speedup vs baseline: 1.9545x; 1.9545x over previous
import functools

import jax
import jax.numpy as jnp
from jax import lax
from jax.experimental import pallas as pl
from jax.experimental.pallas import tpu as pltpu
from jax.experimental.pallas import tpu_sc as plsc

_NUM_EMB = 1024
_DIM = 64
_N = 16384
_COMMIT = 0.25

_ROWS = 256
_GRID = _N // _ROWS

_CHUNK = 128


def _tc_body(x_ref, w_ref, idx_ref, enc_ref, loss_ref, perp_ref,
             counts_acc, sse_acc, se2_acc, w2_acc):
    i = pl.program_id(0)
    x = x_ref[...]

    @pl.when(i == 0)
    def _():
        w = w_ref[...]
        se2_acc[...] = jnp.sum(w * w, axis=1).reshape(1, _NUM_EMB)
        w2_acc[...] = w + w

    sx2 = jnp.sum(x * x, axis=1, keepdims=True)
    se2 = se2_acc[...]
    mm2 = lax.dot_general(x, w2_acc[...], (((1,), (1,)), ((), ())),
                          preferred_element_type=jnp.float32)
    dist = (sx2 + se2) - mm2
    m = jnp.min(dist, axis=1, keepdims=True)
    col = lax.broadcasted_iota(jnp.int32, (_ROWS, _NUM_EMB), 1)
    idx = jnp.min(jnp.where(dist == m, col, _NUM_EMB), axis=1)
    idx_ref[...] = idx.reshape(1, 1, _ROWS)
    one_hot = (col == idx[:, None]).astype(jnp.float32)
    enc_ref[...] = one_hot

    @pl.when(i == 0)
    def _():
        counts_acc[...] = jnp.zeros((1, _NUM_EMB), jnp.float32)
        sse_acc[0, 0] = 0.0

    counts_acc[...] = counts_acc[...] + jnp.sum(one_hot, axis=0,
                                                keepdims=True)
    sse_acc[0, 0] = sse_acc[0, 0] + jnp.sum(m)

    @pl.when(i == _GRID - 1)
    def _():
        avg = counts_acc[...] * (1.0 / _N)
        perp = jnp.exp(-jnp.sum(avg * jnp.log(avg + 1e-10)))
        perp_ref[...] = perp.reshape(1, 1)
        loss_ref[...] = (_COMMIT * (sse_acc[0, 0] / (_N * _DIM))).reshape(1, 1)


_tc_call = pl.pallas_call(
    _tc_body,
    grid=(_GRID,),
    in_specs=[
        pl.BlockSpec((_ROWS, _DIM), lambda i: (i, 0)),
        pl.BlockSpec((_NUM_EMB, _DIM), lambda i: (0, 0)),
    ],
    out_specs=[
        pl.BlockSpec((1, 1, _ROWS), lambda i: (i, 0, 0)),
        pl.BlockSpec((_ROWS, _NUM_EMB), lambda i: (i, 0)),
        pl.BlockSpec((1, 1), lambda i: (0, 0)),
        pl.BlockSpec((1, 1), lambda i: (0, 0)),
    ],
    out_shape=[
        jax.ShapeDtypeStruct((_GRID, 1, _ROWS), jnp.int32),
        jax.ShapeDtypeStruct((_N, _NUM_EMB), jnp.float32),
        jax.ShapeDtypeStruct((1, 1), jnp.float32),
        jax.ShapeDtypeStruct((1, 1), jnp.float32),
    ],
    scratch_shapes=[
        pltpu.VMEM((1, _NUM_EMB), jnp.float32),
        pltpu.SMEM((1, 1), jnp.float32),
        pltpu.VMEM((1, _NUM_EMB), jnp.float32),
        pltpu.VMEM((_NUM_EMB, _DIM), jnp.float32),
    ],
)


def _make_sc_gather():
    info = plsc.get_sparse_core_info()
    nw = info.num_cores * info.num_subcores
    bpw = _N // nw
    k = bpw // _CHUNK

    def body(idx_hbm, w_hbm, out_hbm, idx_v, rows_v, sem):
        wid = lax.axis_index("s") * info.num_cores + lax.axis_index("c")
        base = wid * bpw
        pltpu.sync_copy(idx_hbm.at[wid], idx_v)
        copies = [
            pltpu.async_copy(w_hbm.at[idx_v.at[j]], rows_v.at[j], sem)
            for j in range(k)
        ]
        for c in copies:
            c.wait()
        for j in range(k):
            pltpu.sync_copy(rows_v.at[j],
                            out_hbm.at[pl.ds(base + j * _CHUNK, _CHUNK)])

    return pl.kernel(
        body,
        mesh=plsc.VectorSubcoreMesh(core_axis_name="c", subcore_axis_name="s"),
        out_type=jax.ShapeDtypeStruct((_N, _DIM), jnp.float32),
        scratch_types=[
            pltpu.VMEM((k, _CHUNK), jnp.int32),
            pltpu.VMEM((k, _CHUNK, _DIM), jnp.float32),
            pltpu.SemaphoreType.DMA,
        ],
        compiler_params=pltpu.CompilerParams(use_tc_tiling_on_sc=False),
    ), nw, k


def kernel(inputs, embedding_weight):
    idx3, encodings, loss, perp = _tc_call(inputs, embedding_weight)
    sc_gather, nw, k = _sc_gather_cached
    idx_tiles = idx3.reshape(nw, k, _CHUNK)
    quantized_st = sc_gather(idx_tiles, embedding_weight)
    return (loss.reshape(()), quantized_st, perp.reshape(()), encodings)


_sc_gather_cached = _make_sc_gather()


# speedup vs baseline: 2.1374x; 1.0936x over previous
import functools

import jax
import jax.numpy as jnp
from jax import lax
from jax.experimental import pallas as pl
from jax.experimental.pallas import tpu as pltpu
from jax.experimental.pallas import tpu_sc as plsc

_NUM_EMB = 1024
_DIM = 64
_N = 16384
_COMMIT = 0.25

_ROWS = 512
_GRID = _N // _ROWS

_CHUNK = 128


def _tc_body(x_ref, w_ref, idx_ref, enc_ref, loss_ref, perp_ref,
             counts_acc, sse_acc, se2_acc, w2_acc):
    i = pl.program_id(0)
    x = x_ref[...]

    @pl.when(i == 0)
    def _():
        w = w_ref[...]
        se2_acc[...] = jnp.sum(w * w, axis=1).reshape(1, _NUM_EMB)
        w2_acc[...] = w + w

    sx2 = jnp.sum(x * x, axis=1, keepdims=True)
    se2 = se2_acc[...]
    mm2 = lax.dot_general(x, w2_acc[...], (((1,), (1,)), ((), ())),
                          preferred_element_type=jnp.float32)
    dist = (sx2 + se2) - mm2
    m = jnp.min(dist, axis=1, keepdims=True)
    col = lax.broadcasted_iota(jnp.int32, (_ROWS, _NUM_EMB), 1)
    idx = jnp.min(jnp.where(dist == m, col, _NUM_EMB), axis=1)
    idx_ref[...] = idx.reshape(1, 1, _ROWS)
    one_hot = (col == idx[:, None]).astype(jnp.float32)
    enc_ref[...] = one_hot

    @pl.when(i == 0)
    def _():
        counts_acc[...] = jnp.zeros((1, _NUM_EMB), jnp.float32)
        sse_acc[0, 0] = 0.0

    counts_acc[...] = counts_acc[...] + jnp.sum(one_hot, axis=0,
                                                keepdims=True)
    sse_acc[0, 0] = sse_acc[0, 0] + jnp.sum(m)

    @pl.when(i == _GRID - 1)
    def _():
        avg = counts_acc[...] * (1.0 / _N)
        perp = jnp.exp(-jnp.sum(avg * jnp.log(avg + 1e-10)))
        perp_ref[...] = perp.reshape(1, 1)
        loss_ref[...] = (_COMMIT * (sse_acc[0, 0] / (_N * _DIM))).reshape(1, 1)


_tc_call = pl.pallas_call(
    _tc_body,
    grid=(_GRID,),
    in_specs=[
        pl.BlockSpec((_ROWS, _DIM), lambda i: (i, 0)),
        pl.BlockSpec((_NUM_EMB, _DIM), lambda i: (0, 0)),
    ],
    out_specs=[
        pl.BlockSpec((1, 1, _ROWS), lambda i: (i, 0, 0)),
        pl.BlockSpec((_ROWS, _NUM_EMB), lambda i: (i, 0)),
        pl.BlockSpec((1, 1), lambda i: (0, 0)),
        pl.BlockSpec((1, 1), lambda i: (0, 0)),
    ],
    out_shape=[
        jax.ShapeDtypeStruct((_GRID, 1, _ROWS), jnp.int32),
        jax.ShapeDtypeStruct((_N, _NUM_EMB), jnp.float32),
        jax.ShapeDtypeStruct((1, 1), jnp.float32),
        jax.ShapeDtypeStruct((1, 1), jnp.float32),
    ],
    scratch_shapes=[
        pltpu.VMEM((1, _NUM_EMB), jnp.float32),
        pltpu.SMEM((1, 1), jnp.float32),
        pltpu.VMEM((1, _NUM_EMB), jnp.float32),
        pltpu.VMEM((_NUM_EMB, _DIM), jnp.float32),
    ],
)


def _make_sc_gather():
    info = plsc.get_sparse_core_info()
    nw = info.num_cores * info.num_subcores
    bpw = _N // nw
    k = bpw // _CHUNK

    def body(idx_hbm, w_hbm, out_hbm, idx_v, rows_v, sem):
        wid = lax.axis_index("s") * info.num_cores + lax.axis_index("c")
        base = wid * bpw
        pltpu.sync_copy(idx_hbm.at[wid], idx_v)
        copies = [
            pltpu.async_copy(w_hbm.at[idx_v.at[j]], rows_v.at[j], sem)
            for j in range(k)
        ]
        for c in copies:
            c.wait()
        for j in range(k):
            pltpu.sync_copy(rows_v.at[j],
                            out_hbm.at[pl.ds(base + j * _CHUNK, _CHUNK)])

    return pl.kernel(
        body,
        mesh=plsc.VectorSubcoreMesh(core_axis_name="c", subcore_axis_name="s"),
        out_type=jax.ShapeDtypeStruct((_N, _DIM), jnp.float32),
        scratch_types=[
            pltpu.VMEM((k, _CHUNK), jnp.int32),
            pltpu.VMEM((k, _CHUNK, _DIM), jnp.float32),
            pltpu.SemaphoreType.DMA,
        ],
        compiler_params=pltpu.CompilerParams(use_tc_tiling_on_sc=False),
    ), nw, k


def kernel(inputs, embedding_weight):
    idx3, encodings, loss, perp = _tc_call(inputs, embedding_weight)
    sc_gather, nw, k = _sc_gather_cached
    idx_tiles = idx3.reshape(nw, k, _CHUNK)
    quantized_st = sc_gather(idx_tiles, embedding_weight)
    return (loss.reshape(()), quantized_st, perp.reshape(()), encodings)


_sc_gather_cached = _make_sc_gather()


# speedup vs baseline: 2.2324x; 1.0444x over previous
import functools

import jax
import jax.numpy as jnp
from jax import lax
from jax.experimental import pallas as pl
from jax.experimental.pallas import tpu as pltpu
from jax.experimental.pallas import tpu_sc as plsc

_NUM_EMB = 1024
_DIM = 64
_N = 16384
_COMMIT = 0.25

_ROWS = 1024
_GRID = _N // _ROWS

_CHUNK = 128


def _tc_body(x_ref, w_ref, idx_ref, enc_ref, loss_ref, perp_ref,
             counts_acc, sse_acc, se2_acc, w2_acc):
    i = pl.program_id(0)
    x = x_ref[...]

    @pl.when(i == 0)
    def _():
        w = w_ref[...]
        se2_acc[...] = jnp.sum(w * w, axis=1).reshape(1, _NUM_EMB)
        w2_acc[...] = w + w

    sx2 = jnp.sum(x * x, axis=1, keepdims=True)
    se2 = se2_acc[...]
    mm2 = lax.dot_general(x, w2_acc[...], (((1,), (1,)), ((), ())),
                          preferred_element_type=jnp.float32)
    dist = (sx2 + se2) - mm2
    m = jnp.min(dist, axis=1, keepdims=True)
    col = lax.broadcasted_iota(jnp.int32, (_ROWS, _NUM_EMB), 1)
    idx = jnp.min(jnp.where(dist == m, col, _NUM_EMB), axis=1)
    idx_ref[...] = idx.reshape(1, 1, _ROWS)
    one_hot = (col == idx[:, None]).astype(jnp.float32)
    enc_ref[...] = one_hot

    @pl.when(i == 0)
    def _():
        counts_acc[...] = jnp.zeros((1, _NUM_EMB), jnp.float32)
        sse_acc[0, 0] = 0.0

    counts_acc[...] = counts_acc[...] + jnp.sum(one_hot, axis=0,
                                                keepdims=True)
    sse_acc[0, 0] = sse_acc[0, 0] + jnp.sum(m)

    @pl.when(i == _GRID - 1)
    def _():
        avg = counts_acc[...] * (1.0 / _N)
        perp = jnp.exp(-jnp.sum(avg * jnp.log(avg + 1e-10)))
        perp_ref[...] = perp.reshape(1, 1)
        loss_ref[...] = (_COMMIT * (sse_acc[0, 0] / (_N * _DIM))).reshape(1, 1)


_tc_call = pl.pallas_call(
    _tc_body,
    grid=(_GRID,),
    in_specs=[
        pl.BlockSpec((_ROWS, _DIM), lambda i: (i, 0)),
        pl.BlockSpec((_NUM_EMB, _DIM), lambda i: (0, 0)),
    ],
    out_specs=[
        pl.BlockSpec((1, 1, _ROWS), lambda i: (i, 0, 0)),
        pl.BlockSpec((_ROWS, _NUM_EMB), lambda i: (i, 0)),
        pl.BlockSpec((1, 1), lambda i: (0, 0)),
        pl.BlockSpec((1, 1), lambda i: (0, 0)),
    ],
    out_shape=[
        jax.ShapeDtypeStruct((_GRID, 1, _ROWS), jnp.int32),
        jax.ShapeDtypeStruct((_N, _NUM_EMB), jnp.float32),
        jax.ShapeDtypeStruct((1, 1), jnp.float32),
        jax.ShapeDtypeStruct((1, 1), jnp.float32),
    ],
    scratch_shapes=[
        pltpu.VMEM((1, _NUM_EMB), jnp.float32),
        pltpu.SMEM((1, 1), jnp.float32),
        pltpu.VMEM((1, _NUM_EMB), jnp.float32),
        pltpu.VMEM((_NUM_EMB, _DIM), jnp.float32),
    ],
)


def _make_sc_gather():
    info = plsc.get_sparse_core_info()
    nw = info.num_cores * info.num_subcores
    bpw = _N // nw
    k = bpw // _CHUNK

    def body(idx_hbm, w_hbm, out_hbm, idx_v, rows_v, sem):
        wid = lax.axis_index("s") * info.num_cores + lax.axis_index("c")
        base = wid * bpw
        pltpu.sync_copy(idx_hbm.at[wid], idx_v)
        copies = [
            pltpu.async_copy(w_hbm.at[idx_v.at[j]], rows_v.at[j], sem)
            for j in range(k)
        ]
        for c in copies:
            c.wait()
        for j in range(k):
            pltpu.sync_copy(rows_v.at[j],
                            out_hbm.at[pl.ds(base + j * _CHUNK, _CHUNK)])

    return pl.kernel(
        body,
        mesh=plsc.VectorSubcoreMesh(core_axis_name="c", subcore_axis_name="s"),
        out_type=jax.ShapeDtypeStruct((_N, _DIM), jnp.float32),
        scratch_types=[
            pltpu.VMEM((k, _CHUNK), jnp.int32),
            pltpu.VMEM((k, _CHUNK, _DIM), jnp.float32),
            pltpu.SemaphoreType.DMA,
        ],
        compiler_params=pltpu.CompilerParams(use_tc_tiling_on_sc=False),
    ), nw, k


def kernel(inputs, embedding_weight):
    idx3, encodings, loss, perp = _tc_call(inputs, embedding_weight)
    sc_gather, nw, k = _sc_gather_cached
    idx_tiles = idx3.reshape(nw, k, _CHUNK)
    quantized_st = sc_gather(idx_tiles, embedding_weight)
    return (loss.reshape(()), quantized_st, perp.reshape(()), encodings)


_sc_gather_cached = _make_sc_gather()
